# Initial kernel scaffold; baseline (speedup 1.0000x reference)
import jax
import jax.numpy as jnp
from jax.experimental import pallas as pl


def kernel(data, table):
    raise NotImplementedError("write your pallas kernel here")


# speedup vs baseline: 1.7957x; 1.7957x over previous
import functools

import jax
import jax.numpy as jnp
from jax import lax
from jax.experimental import pallas as pl
from jax.experimental.pallas import tpu as pltpu
from jax.experimental.pallas import tpu_sc as plsc

D_MODEL = 64

_info = plsc.get_sparse_core_info()
_NC, _NS = _info.num_cores, _info.num_subcores
_NW = _NC * _NS

_CHUNK = 512


def _gather_call(table, idx):
    B = idx.shape[0]
    b_per_w = B // _NW
    n_chunks = b_per_w // _CHUNK
    mesh = plsc.VectorSubcoreMesh(core_axis_name="c", subcore_axis_name="s")

    @functools.partial(
        pl.kernel,
        mesh=mesh,
        out_type=jax.ShapeDtypeStruct((B, D_MODEL), jnp.float32),
        scratch_types=[
            pltpu.VMEM((_CHUNK,), jnp.int32),
            pltpu.VMEM((_CHUNK, D_MODEL), jnp.float32),
            pltpu.SemaphoreType.DMA,
        ],
        compiler_params=pltpu.CompilerParams(use_tc_tiling_on_sc=False),
    )
    def k(table_hbm, idx_hbm, out_hbm, idx_v, rows_v, sem):
        wid = lax.axis_index("s") * _NC + lax.axis_index("c")
        base = wid * b_per_w

        def body(j, carry):
            off = base + j * _CHUNK
            pltpu.sync_copy(idx_hbm.at[pl.ds(off, _CHUNK)], idx_v)
            pltpu.async_copy(table_hbm.at[idx_v], rows_v, sem).wait()
            pltpu.sync_copy(rows_v, out_hbm.at[pl.ds(off, _CHUNK)])
            return carry

        lax.fori_loop(0, n_chunks, body, 0)

    return k(table, idx)


def kernel(data, table):
    idx = data.reshape(-1)
    out = _gather_call(table, idx)
    return out.reshape(data.shape + (D_MODEL,))


# speedup vs baseline: 1.8902x; 1.0526x over previous
import functools

import jax
import jax.numpy as jnp
from jax import lax
from jax.experimental import pallas as pl
from jax.experimental.pallas import tpu as pltpu
from jax.experimental.pallas import tpu_sc as plsc

D_MODEL = 64

_info = plsc.get_sparse_core_info()
_NC, _NS = _info.num_cores, _info.num_subcores
_NW = _NC * _NS

_CHUNK = 512


def _gather_call(table, idx):
    B = idx.shape[0]
    b_per_w = B // _NW
    n = b_per_w // _CHUNK
    assert n >= 3 and n % 2 == 0
    mesh = plsc.VectorSubcoreMesh(core_axis_name="c", subcore_axis_name="s")

    @functools.partial(
        pl.kernel,
        mesh=mesh,
        out_type=jax.ShapeDtypeStruct((B, D_MODEL), jnp.float32),
        scratch_types=[
            pltpu.VMEM((_CHUNK,), jnp.int32),
            pltpu.VMEM((_CHUNK,), jnp.int32),
            pltpu.VMEM((_CHUNK, D_MODEL), jnp.float32),
            pltpu.VMEM((_CHUNK, D_MODEL), jnp.float32),
            pltpu.SemaphoreType.DMA,
            pltpu.SemaphoreType.DMA,
            pltpu.SemaphoreType.DMA,
            pltpu.SemaphoreType.DMA,
            pltpu.SemaphoreType.DMA,
            pltpu.SemaphoreType.DMA,
        ],
        compiler_params=pltpu.CompilerParams(use_tc_tiling_on_sc=False),
    )
    def k(table_hbm, idx_hbm, out_hbm,
          idx0, idx1, rows0, rows1, si0, si1, sg0, sg1, so0, so1):
        idx_v = (idx0, idx1)
        rows_v = (rows0, rows1)
        s_i = (si0, si1)
        s_g = (sg0, sg1)
        s_o = (so0, so1)
        wid = lax.axis_index("s") * _NC + lax.axis_index("c")
        base = wid * b_per_w

        def idx_copy(g, b):
            return pltpu.make_async_copy(
                idx_hbm.at[pl.ds(base + g * _CHUNK, _CHUNK)], idx_v[b], s_i[b])

        def gather_copy(b):
            return pltpu.make_async_copy(
                table_hbm.at[idx_v[b]], rows_v[b], s_g[b])

        def out_copy(g, b):
            return pltpu.make_async_copy(
                rows_v[b], out_hbm.at[pl.ds(base + g * _CHUNK, _CHUNK)], s_o[b])

        idx_copy(0, 0).start()
        idx_copy(1, 1).start()
        idx_copy(0, 0).wait()
        gather_copy(0).start()
        idx_copy(1, 1).wait()
        gather_copy(1).start()
        gather_copy(0).wait()
        out_copy(0, 0).start()
        idx_copy(2, 0).start()

        def pair(i, carry):
            for b in (0, 1):
                g = 2 + 2 * i + b
                pb = 1 - b
                idx_copy(g, b).wait()
                out_copy(g - 2, b).wait()
                gather_copy(b).start()
                gather_copy(pb).wait()
                out_copy(g - 1, pb).start()

                @pl.when(g + 1 < n)
                def _():
                    idx_copy(g + 1, pb).start()

            return carry

        lax.fori_loop(0, (n - 2) // 2, pair, 0)

        lb = (n - 1) % 2
        gather_copy(lb).wait()
        out_copy(n - 1, lb).start()
        out_copy(n - 2, 1 - lb).wait()
        out_copy(n - 1, lb).wait()

    return k(table, idx)


def kernel(data, table):
    idx = data.reshape(-1)
    out = _gather_call(table, idx)
    return out.reshape(data.shape + (D_MODEL,))
